# baseline (speedup 1.0000x reference)
import functools

import jax
import jax.numpy as jnp
from jax import lax
from jax.experimental import pallas as pl
from jax.experimental.pallas import tpu as pltpu
from jax.experimental.pallas import tpu_sc as plsc

NC = 2
NS = 16
NW = NC * NS
K = 80
L = 16


@functools.lru_cache(maxsize=None)
def _sc_segsum(n, e, d, with_counts):
    epw = e // NW
    nchunk = epw // K
    rquot = 8 * (n // (NS * 8))
    tail = n - NS * rquot
    assert epw % K == 0 and tail % 8 == 0 and tail <= rquot and n % L == 0

    NBUF = 3 if with_counts else 4
    NQ = 2 * NBUF

    mesh = plsc.VectorSubcoreMesh(core_axis_name="c", subcore_axis_name="s")
    out_type = [jax.ShapeDtypeStruct((NC, n, d), jnp.float32)]
    scratch = {
        "src_c": pltpu.VMEM((NQ, K), jnp.int32),
        "dst_b": pltpu.VMEM((NQ, K), jnp.int32),
        "acc_s": pltpu.VMEM_SHARED((n, d), jnp.float32),
    }
    for b in range(NBUF):
        scratch[f"rows{b}"] = pltpu.VMEM((K, d), jnp.float32)
        scratch[f"gsem{b}"] = pltpu.SemaphoreType.DMA
        scratch[f"ssem{b}"] = pltpu.SemaphoreType.DMA
    for q in range(NQ):
        scratch[f"isem{q}"] = pltpu.SemaphoreType.DMA
    if with_counts:
        out_type.append(jax.ShapeDtypeStruct((NW * n,), jnp.float32))
        scratch["cnt_v"] = pltpu.VMEM((n,), jnp.float32)

    def body(x_hbm, src_hbm, dst_hbm, zeros_hbm, sums_hbm, cnts_hbm=None,
             *, src_c, dst_b, acc_s, cnt_v=None, **scr):
        c = lax.axis_index("c")
        s = lax.axis_index("s")
        w = c * NS + s
        rbase = s * rquot
        ebase = w * epw

        def over_rows(fn):
            fn(rbase, rquot)
            if tail:
                @pl.when(s == NS - 1)
                def _():
                    fn(NS * rquot, tail)

        over_rows(lambda b, m: pltpu.sync_copy(
            zeros_hbm.at[pl.ds(0, m)], acc_s.at[pl.ds(b, m)]))
        if with_counts:
            def zero_cnt(i, carry):
                cnt_v[pl.ds(i * L, L)] = jnp.zeros((L,), jnp.float32)
                return carry
            lax.fori_loop(0, n // L, zero_cnt, 0)
        plsc.subcore_barrier()

        ones16 = jnp.ones((L,), jnp.float32)
        rows = tuple(scr[f"rows{b}"] for b in range(NBUF))
        gsems = tuple(scr[f"gsem{b}"] for b in range(NBUF))
        ssems = tuple(scr[f"ssem{b}"] for b in range(NBUF))
        isems = tuple(scr[f"isem{q}"] for q in range(NQ))

        def idx_load(i, q):
            off = ebase + i * K
            pltpu.async_copy(src_hbm.at[pl.ds(off, K)], src_c.at[q],
                             isems[q])
            pltpu.async_copy(dst_hbm.at[pl.ds(off, K)], dst_b.at[q],
                             isems[q])

        def iwait(i, q):
            off = ebase + i * K
            pltpu.make_async_copy(src_hbm.at[pl.ds(off, K)], src_c.at[q],
                                  isems[q]).wait()
            pltpu.make_async_copy(dst_hbm.at[pl.ds(off, K)], dst_b.at[q],
                                  isems[q]).wait()

        def gfire(p, q):
            pltpu.async_copy(x_hbm.at[src_c.at[q]], rows[p], gsems[p])

        def gwait(p, q):
            pltpu.make_async_copy(x_hbm.at[src_c.at[q]], rows[p],
                                  gsems[p]).wait()

        def sfire(p, q):
            pltpu.async_copy(rows[p], acc_s.at[dst_b.at[q]], ssems[p],
                             add=True)

        def swaitf(p, q):
            pltpu.make_async_copy(rows[p], acc_s.at[dst_b.at[q]],
                                  ssems[p]).wait()

        def counts(q):
            if with_counts:
                for j in range(K // L):
                    idx = dst_b[q, pl.ds(j * L, L)]
                    plsc.addupdate_scatter(cnt_v, [idx], ones16)

        def substep(i, k, head=False, fire_g=True, fire_i=True):
            p, pn = k % NBUF, (k + 1) % NBUF
            q, qn, q2 = k % NQ, (k + 1) % NQ, (k + 2) % NQ
            if not head:
                swaitf(pn, (k + 1 + NBUF) % NQ)
            if fire_g:
                iwait(i + 1, qn)
                gfire(pn, qn)
            if fire_i:
                idx_load(i + 2, q2)
            gwait(p, q)
            sfire(p, q)
            counts(q)

        S = NBUF - 1
        NLOOP = (nchunk - S) // NQ
        idx_load(0, 0)
        iwait(0, 0)
        gfire(0, 0)
        idx_load(1, 1)
        for i in range(S):
            substep(i, i, head=True)

        def steady(t, carry):
            i0 = NQ * t + S
            for k in range(NQ):
                substep(i0 + k, (S + k) % NQ)
            return carry

        lax.fori_loop(0, NLOOP, steady, 0)
        for i in range(S + NQ * NLOOP, nchunk):
            substep(i, i % NQ, fire_g=(i + 1 < nchunk),
                    fire_i=(i + 2 < nchunk))
        for j in range(nchunk - S, nchunk):
            swaitf(j % NBUF, j % NQ)
        plsc.subcore_barrier()
        over_rows(lambda b, m: pltpu.sync_copy(
            acc_s.at[pl.ds(b, m)], sums_hbm.at[c].at[pl.ds(b, m)]))
        if with_counts:
            pltpu.sync_copy(cnt_v, cnts_hbm.at[pl.ds(w * n, n)])

    if with_counts:
        def body_wc(x, src, dst, z, sums, cnts, **scr):
            body(x, src, dst, z, sums, cnts, **scr)
        fn = body_wc
    else:
        def body_nc(x, src, dst, z, sums, **scr):
            body(x, src, dst, z, sums, None, **scr)
        fn = body_nc

    return pl.kernel(
        fn, out_type=out_type, mesh=mesh, scratch_types=scratch,
        compiler_params=pltpu.CompilerParams(needs_layout_passes=False))


def _tc1_body(sums_ref, cnts_ref, x_ref, wl_ref, bl_ref, wr_ref, g_ref,
              b_ref, o_ref, cnt_ref):
    cnt = jnp.maximum(jnp.sum(cnts_ref[...], axis=0), 1.0)[:, None]
    cnt_ref[...] = cnt
    ssum = sums_ref[0] + sums_ref[1]
    mean = ssum / cnt
    h = (jnp.dot(mean, wl_ref[...], preferred_element_type=jnp.float32)
         + bl_ref[...]
         + jnp.dot(x_ref[...], wr_ref[...], preferred_element_type=jnp.float32))
    mu = jnp.mean(h, axis=0, keepdims=True)
    var = jnp.mean((h - mu) ** 2, axis=0, keepdims=True)
    hn = (h - mu) * lax.rsqrt(var + 1e-5) * g_ref[...] + b_ref[...]
    o_ref[...] = jnp.where(hn >= 0, hn, 0.01 * hn)


def _tc2_body(sums_ref, cnt_ref, h_ref, wl_ref, bl_ref, wr_ref, o_ref):
    ssum = sums_ref[0] + sums_ref[1]
    mean = ssum / cnt_ref[...]
    o_ref[...] = (jnp.dot(mean, wl_ref[...], preferred_element_type=jnp.float32)
                  + bl_ref[...]
                  + jnp.dot(h_ref[...], wr_ref[...],
                            preferred_element_type=jnp.float32))


def kernel(x, edge_index, Wl1, bl1, Wr1, gamma, beta, Wl2, bl2, Wr2):
    n, d = x.shape
    e = edge_index.shape[1]
    src = edge_index[0].astype(jnp.int32)
    dst = edge_index[1].astype(jnp.int32)
    rquot = 8 * (n // (NS * 8))
    zeros = jnp.zeros((rquot, d), jnp.float32)

    sums1, cnts = _sc_segsum(n, e, d, True)(x, src, dst, zeros)
    h, cnt_col = pl.pallas_call(
        _tc1_body,
        out_shape=[jax.ShapeDtypeStruct((n, d), jnp.float32),
                   jax.ShapeDtypeStruct((n, 1), jnp.float32)],
    )(sums1, cnts.reshape(NW, n), x, Wl1, bl1.reshape(1, -1), Wr1,
      gamma.reshape(1, -1), beta.reshape(1, -1))
    (sums2,) = _sc_segsum(n, e, d, False)(h, src, dst, zeros)
    out = pl.pallas_call(
        _tc2_body,
        out_shape=jax.ShapeDtypeStruct((n, d), jnp.float32),
    )(sums2, cnt_col, h, Wl2, bl2.reshape(1, -1), Wr2)
    return out
